# baseline (speedup 1.0000x reference)
import functools

import jax
import jax.numpy as jnp
from jax import lax
from jax.experimental import pallas as pl
from jax.experimental.pallas import tpu as pltpu
from jax.experimental.pallas import tpu_sc as plsc

_BATCH = 1024
_SEQ = 200
_PQ_M = 8
_VALS = 256
_SUB = 16

_N = _BATCH * _SEQ
_NC, _NS = 2, 16
_NW = _NC * _NS
_TOK_W = _N // _NW
_C = 256
_NCHUNK = _TOK_W // _C


def _body(ids_hbm, codes_hbm, cent_hbm, out_hbm, ids_v, codes_v, cidx_v,
          out_v, sem_c, sem_g, sem_w):
    wid = lax.axis_index("s") * _NC + lax.axis_index("c")
    base_tok = wid * _TOK_W
    lane = jnp.arange(16, dtype=jnp.int32)

    def prefetch(c, buf):
        tok0 = base_tok + c * _C
        pltpu.sync_copy(ids_hbm.at[pl.ds(tok0, _C)], ids_v.at[buf])
        pltpu.async_copy(codes_hbm.at[ids_v.at[buf]], codes_v.at[buf],
                         sem_c)

    def wait_codes():
        pltpu.make_async_copy(codes_hbm.at[ids_v.at[0]], codes_v.at[0],
                              sem_c).wait()

    prefetch(0, 0)

    def chunk(c, _):
        buf = c & 1
        tok0 = base_tok + c * _C

        wait_codes()

        @pl.when(c + 1 < _NCHUNK)
        def _():
            prefetch(c + 1, 1 - buf)

        @pl.when(c >= 2)
        def _():
            for d in range(_PQ_M):
                pltpu.make_async_copy(
                    out_v.at[buf, 0],
                    out_hbm.at[pl.ds(0, _C), pl.ds(0, _SUB)],
                    sem_w.at[buf],
                ).wait()

        def cidx_row(j, _):
            t0 = j * 16
            rows = t0 + lane
            for d in range(_PQ_M):
                code16 = plsc.load_gather(
                    codes_v.at[buf],
                    [rows, jnp.full((16,), d, jnp.int32)])
                cidx_v[d, pl.ds(t0, 16)] = code16 + (d << 8)
            return _

        lax.fori_loop(0, _C // 16, cidx_row, None, unroll=True)

        for d in range(_PQ_M):
            pltpu.async_copy(
                cent_hbm.at[cidx_v.at[d]],
                out_v.at[buf, d],
                sem_g,
            )
        for d in range(_PQ_M):
            pltpu.make_async_copy(
                cent_hbm.at[cidx_v.at[0]],
                out_v.at[0, 0],
                sem_g,
            ).wait()

        for d in range(_PQ_M):
            pltpu.async_copy(
                out_v.at[buf, d],
                out_hbm.at[pl.ds(tok0, _C), pl.ds(d * _SUB, _SUB)],
                sem_w.at[buf],
            )
        return _

    lax.fori_loop(0, _NCHUNK, chunk, None)

    for b in range(2):
        for d in range(_PQ_M):
            pltpu.make_async_copy(out_v.at[b, 0],
                                  out_hbm.at[pl.ds(0, _C), pl.ds(0, _SUB)],
                                  sem_w.at[b]).wait()


@functools.partial(jax.jit, static_argnames=())
def kernel(input_ids, item_codes, centroids):
    ids1 = input_ids.reshape(_N)
    cent = centroids.reshape(_PQ_M * _VALS, _SUB)
    run = pl.kernel(
        _body,
        out_type=jax.ShapeDtypeStruct((_N, _PQ_M * _SUB), jnp.float32),
        mesh=plsc.VectorSubcoreMesh(
            core_axis_name="c", subcore_axis_name="s",
            num_cores=_NC, num_subcores=_NS,
        ),
        scratch_types=[
            pltpu.VMEM((2, _C), jnp.int32),
            pltpu.VMEM((2, _C, _PQ_M), jnp.int32),
            pltpu.VMEM((_PQ_M, _C), jnp.int32),
            pltpu.VMEM((2, _PQ_M, _C, _SUB), jnp.float32),
            pltpu.SemaphoreType.DMA,
            pltpu.SemaphoreType.DMA,
            pltpu.SemaphoreType.DMA((2,)),
        ],
        compiler_params=pltpu.CompilerParams(use_tc_tiling_on_sc=False,
                                             needs_layout_passes=False),
    )
    out = run(ids1, item_codes, cent)
    return out.reshape(_BATCH, _SEQ, _PQ_M * _SUB)
